# baseline (speedup 1.0000x reference)
import functools

import jax
import jax.numpy as jnp
import numpy as np
from jax import lax
from jax.experimental import pallas as pl
from jax.experimental.pallas import tpu as pltpu
from jax.experimental.pallas import tpu_sc as plsc

B = 4096
L = 200
EMB = 64
HID = 256
NCLS = 10
POOL_W = 3 * EMB

_NC = 2
_NS = 16
_NW = _NC * _NS
_RW = B // _NW
_CHUNKS = ((0, 128), (128, 72))
_UNROLL = 8

_COL_PERM = np.concatenate([
    np.arange(0, 32, 2), np.arange(1, 32, 2),
    np.arange(32, 64, 2), np.arange(33, 64, 2),
])
_W1_PERM = np.concatenate([_COL_PERM + 64 * t for t in range(3)])


def _sc_pool_one(x_flat, tab):
    mesh = plsc.VectorSubcoreMesh(core_axis_name="c", subcore_axis_name="s")

    @functools.partial(
        pl.kernel,
        mesh=mesh,
        compiler_params=pltpu.CompilerParams(use_tc_tiling_on_sc=False,
                                             needs_layout_passes=False),
        out_type=jax.ShapeDtypeStruct((B * EMB,), jnp.float32),
        scratch_types=[
            pltpu.VMEM((_RW * L,), jnp.int32),
            pltpu.VMEM((L, EMB // 2), jnp.int32),
            pltpu.VMEM((L, EMB // 2), jnp.int32),
            pltpu.VMEM((EMB,), jnp.float32),
            pltpu.SemaphoreType.DMA,
            pltpu.SemaphoreType.DMA,
        ],
    )
    def pool_kernel(x_hbm, tab_hbm, out, idx_v, rows_a, rows_b, acc_v,
                    sem_a, sem_b):
        wid = lax.axis_index("s") * _NC + lax.axis_index("c")
        base = wid * _RW

        pltpu.sync_copy(x_hbm.at[pl.ds(base * L, _RW * L)], idx_v)

        def mk_copies(i, rbuf, sem):
            return [
                pltpu.make_async_copy(
                    tab_hbm.at[idx_v.at[pl.ds(i * L + o, sz)]],
                    rbuf.at[pl.ds(o, sz), :],
                    sem,
                )
                for o, sz in _CHUNKS
            ]

        def fire(i, rbuf, sem):
            for cp in mk_copies(i, rbuf, sem):
                cp.start()

        def drain(i, rbuf, sem):
            for cp in mk_copies(i, rbuf, sem):
                cp.wait()

        def accum_store(i, rbuf):
            def body(k, accs):
                accs = list(accs)
                for u in range(_UNROLL):
                    r = _UNROLL * k + u
                    for c in range(2):
                        v = rbuf[r, pl.ds(16 * c, 16)]
                        a = plsc.bitcast(v << 16, jnp.float32)
                        b = plsc.bitcast(
                            v & jnp.int32(-65536), jnp.float32)
                        accs[2 * c] = accs[2 * c] + a
                        accs[2 * c + 1] = accs[2 * c + 1] + b
                return tuple(accs)

            z = jnp.zeros((16,), jnp.float32)
            accs = lax.fori_loop(0, L // _UNROLL, body, (z, z, z, z))
            for c in range(4):
                acc_v[pl.ds(16 * c, 16)] = accs[c]
            pltpu.sync_copy(acc_v, out.at[pl.ds((base + i) * EMB, EMB)])

        fire(0, rows_a, sem_a)

        def pair_body(j, _):
            i0 = 2 * j
            fire(i0 + 1, rows_b, sem_b)
            drain(i0, rows_a, sem_a)
            accum_store(i0, rows_a)

            @pl.when(j < _RW // 2 - 1)
            def _():
                fire(i0 + 2, rows_a, sem_a)

            drain(i0 + 1, rows_b, sem_b)
            accum_store(i0 + 1, rows_b)
            return 0

        lax.fori_loop(0, _RW // 2, pair_body, 0)

    return pool_kernel(x_flat, tab)


def _mlp_body(p1_ref, p2_ref, p3_ref, w1_ref, b1_ref, w2_ref, b2_ref, o_ref):
    h = jnp.dot(p1_ref[...], w1_ref[pl.ds(0, EMB), :],
                preferred_element_type=jnp.float32)
    h = h + jnp.dot(p2_ref[...], w1_ref[pl.ds(EMB, EMB), :],
                    preferred_element_type=jnp.float32)
    h = h + jnp.dot(p3_ref[...], w1_ref[pl.ds(2 * EMB, EMB), :],
                    preferred_element_type=jnp.float32)
    h = h * (1.0 / L) + b1_ref[...]
    h = jnp.maximum(h, 0.0)
    o = jnp.dot(h, w2_ref[...], preferred_element_type=jnp.float32)
    o_ref[...] = o + b2_ref[...]


def _tc_mlp(p1, p2, p3, W1p, b1, W2, b2):
    blk = 512
    grid = (B // blk,)
    return pl.pallas_call(
        _mlp_body,
        grid=grid,
        in_specs=[
            pl.BlockSpec((blk, EMB), lambda i: (i, 0)),
            pl.BlockSpec((blk, EMB), lambda i: (i, 0)),
            pl.BlockSpec((blk, EMB), lambda i: (i, 0)),
            pl.BlockSpec((POOL_W, HID), lambda i: (0, 0)),
            pl.BlockSpec((1, HID), lambda i: (0, 0)),
            pl.BlockSpec((HID, NCLS), lambda i: (0, 0)),
            pl.BlockSpec((1, NCLS), lambda i: (0, 0)),
        ],
        out_specs=pl.BlockSpec((blk, NCLS), lambda i: (i, 0)),
        out_shape=jax.ShapeDtypeStruct((B, NCLS), jnp.float32),
    )(p1, p2, p3, W1p, b1.reshape(1, HID), W2, b2.reshape(1, NCLS))


def kernel(x_word, x_bigram, x_trigram, emb_word, emb_bigram, emb_trigram,
           W1, b1, W2, b2):
    pools = []
    for x, tab in ((x_word, emb_word), (x_bigram, emb_bigram),
                   (x_trigram, emb_trigram)):
        tab_i32 = jax.lax.bitcast_convert_type(
            tab.astype(jnp.bfloat16).reshape(tab.shape[0], EMB // 2, 2),
            jnp.int32)
        pools.append(_sc_pool_one(x.reshape(B * L), tab_i32))
    p1, p2, p3 = (p.reshape(B, EMB) for p in pools)
    return _tc_mlp(p1, p2, p3, W1[_W1_PERM, :], b1, W2, b2)


# speedup vs baseline: 2.0474x; 2.0474x over previous
import functools

import jax
import jax.numpy as jnp
import numpy as np
from jax import lax
from jax.experimental import pallas as pl
from jax.experimental.pallas import tpu as pltpu
from jax.experimental.pallas import tpu_sc as plsc

B = 4096
L = 200
EMB = 64
HID = 256
NCLS = 10
POOL_W = 3 * EMB

_NC = 2
_NS = 16
_NW = _NC * _NS
_RW = B // _NW
_CHUNKS = ((0, 128), (128, 72))
_UNROLL = 8

_COL_PERM = np.concatenate([
    np.arange(0, 32, 2), np.arange(1, 32, 2),
    np.arange(32, 64, 2), np.arange(33, 64, 2),
])
_W1_PERM = np.concatenate([_COL_PERM + 64 * t for t in range(3)])


def _sc_pool_one(x_flat, tab):
    mesh = plsc.VectorSubcoreMesh(core_axis_name="c", subcore_axis_name="s")

    @functools.partial(
        pl.kernel,
        mesh=mesh,
        compiler_params=pltpu.CompilerParams(use_tc_tiling_on_sc=False,
                                             needs_layout_passes=False),
        out_type=jax.ShapeDtypeStruct((B * EMB,), jnp.float32),
        scratch_types=[
            pltpu.VMEM((_RW * L,), jnp.int32),
            pltpu.VMEM((L, EMB), jnp.bfloat16),
            pltpu.VMEM((L, EMB), jnp.bfloat16),
            pltpu.VMEM((EMB,), jnp.float32),
            pltpu.SemaphoreType.DMA,
            pltpu.SemaphoreType.DMA,
        ],
    )
    def pool_kernel(x_hbm, tab_hbm, out, idx_v, rows_a, rows_b, acc_v,
                    sem_a, sem_b):
        wid = lax.axis_index("s") * _NC + lax.axis_index("c")
        base = wid * _RW

        pltpu.sync_copy(x_hbm.at[pl.ds(base * L, _RW * L)], idx_v)

        def mk_copies(i, rbuf, sem):
            return [
                pltpu.make_async_copy(
                    tab_hbm.at[idx_v.at[pl.ds(i * L + o, sz)]],
                    rbuf.at[pl.ds(o, sz), :],
                    sem,
                )
                for o, sz in _CHUNKS
            ]

        def fire(i, rbuf, sem):
            for cp in mk_copies(i, rbuf, sem):
                cp.start()

        def drain(i, rbuf, sem):
            for cp in mk_copies(i, rbuf, sem):
                cp.wait()

        def accum_store(i, rbuf):
            def body(k, accs):
                accs = list(accs)
                for u in range(_UNROLL):
                    r = _UNROLL * k + u
                    for c in range(2):
                        v = rbuf[r, pl.ds(32 * c, 32)]
                        a, b = plsc.unpack(v, format=plsc.PackFormat.INTERLEAVED)
                        accs[2 * c] = accs[2 * c] + a.astype(jnp.float32)
                        accs[2 * c + 1] = accs[2 * c + 1] + b.astype(jnp.float32)
                return tuple(accs)

            z = jnp.zeros((16,), jnp.float32)
            accs = lax.fori_loop(0, L // _UNROLL, body, (z, z, z, z))
            for c in range(4):
                acc_v[pl.ds(16 * c, 16)] = accs[c]
            pltpu.sync_copy(acc_v, out.at[pl.ds((base + i) * EMB, EMB)])

        fire(0, rows_a, sem_a)

        def pair_body(j, _):
            i0 = 2 * j
            fire(i0 + 1, rows_b, sem_b)
            drain(i0, rows_a, sem_a)
            accum_store(i0, rows_a)

            @pl.when(j < _RW // 2 - 1)
            def _():
                fire(i0 + 2, rows_a, sem_a)

            drain(i0 + 1, rows_b, sem_b)
            accum_store(i0 + 1, rows_b)
            return 0

        lax.fori_loop(0, _RW // 2, pair_body, 0)

    return pool_kernel(x_flat, tab)


def _mlp_body(p1_ref, p2_ref, p3_ref, w1_ref, b1_ref, w2_ref, b2_ref, o_ref):
    h = jnp.dot(p1_ref[...], w1_ref[pl.ds(0, EMB), :],
                preferred_element_type=jnp.float32)
    h = h + jnp.dot(p2_ref[...], w1_ref[pl.ds(EMB, EMB), :],
                    preferred_element_type=jnp.float32)
    h = h + jnp.dot(p3_ref[...], w1_ref[pl.ds(2 * EMB, EMB), :],
                    preferred_element_type=jnp.float32)
    h = h * (1.0 / L) + b1_ref[...]
    h = jnp.maximum(h, 0.0)
    o = jnp.dot(h, w2_ref[...], preferred_element_type=jnp.float32)
    o_ref[...] = o + b2_ref[...]


def _tc_mlp(p1, p2, p3, W1p, b1, W2, b2):
    blk = 512
    grid = (B // blk,)
    return pl.pallas_call(
        _mlp_body,
        grid=grid,
        in_specs=[
            pl.BlockSpec((blk, EMB), lambda i: (i, 0)),
            pl.BlockSpec((blk, EMB), lambda i: (i, 0)),
            pl.BlockSpec((blk, EMB), lambda i: (i, 0)),
            pl.BlockSpec((POOL_W, HID), lambda i: (0, 0)),
            pl.BlockSpec((1, HID), lambda i: (0, 0)),
            pl.BlockSpec((HID, NCLS), lambda i: (0, 0)),
            pl.BlockSpec((1, NCLS), lambda i: (0, 0)),
        ],
        out_specs=pl.BlockSpec((blk, NCLS), lambda i: (i, 0)),
        out_shape=jax.ShapeDtypeStruct((B, NCLS), jnp.float32),
    )(p1, p2, p3, W1p, b1.reshape(1, HID), W2, b2.reshape(1, NCLS))


def kernel(x_word, x_bigram, x_trigram, emb_word, emb_bigram, emb_trigram,
           W1, b1, W2, b2):
    pools = []
    for x, tab in ((x_word, emb_word), (x_bigram, emb_bigram),
                   (x_trigram, emb_trigram)):
        pools.append(_sc_pool_one(x.reshape(B * L), tab.astype(jnp.bfloat16)))
    p1, p2, p3 = (p.reshape(B, EMB) for p in pools)
    return _tc_mlp(p1, p2, p3, W1[_W1_PERM, :], b1, W2, b2)


# speedup vs baseline: 2.7503x; 1.3433x over previous
import functools

import jax
import jax.numpy as jnp
from jax import lax
from jax.experimental import pallas as pl
from jax.experimental.pallas import tpu as pltpu
from jax.experimental.pallas import tpu_sc as plsc

B = 4096
L = 200
EMB = 64
HID = 256
NCLS = 10
POOL_W = 3 * EMB

_NC = 2
_NS = 16
_NW = _NC * _NS
_RW = B // _NW
_CHUNKS = ((0, 128), (128, 72))
_UNROLL = 8

def _sc_pool_one(x_flat, tab):
    mesh = plsc.VectorSubcoreMesh(core_axis_name="c", subcore_axis_name="s")

    @functools.partial(
        pl.kernel,
        mesh=mesh,
        compiler_params=pltpu.CompilerParams(use_tc_tiling_on_sc=False),
        out_type=jax.ShapeDtypeStruct((B * EMB,), jnp.float32),
        scratch_types=[
            pltpu.VMEM((_RW * L,), jnp.int32),
            pltpu.VMEM((L, EMB), jnp.float32),
            pltpu.VMEM((L, EMB), jnp.float32),
            pltpu.VMEM((EMB,), jnp.float32),
            pltpu.SemaphoreType.DMA,
            pltpu.SemaphoreType.DMA,
        ],
    )
    def pool_kernel(x_hbm, tab_hbm, out, idx_v, rows_a, rows_b, acc_v,
                    sem_a, sem_b):
        wid = lax.axis_index("s") * _NC + lax.axis_index("c")
        base = wid * _RW

        pltpu.sync_copy(x_hbm.at[pl.ds(base * L, _RW * L)], idx_v)

        def mk_copies(i, rbuf, sem):
            return [
                pltpu.make_async_copy(
                    tab_hbm.at[idx_v.at[pl.ds(i * L + o, sz)]],
                    rbuf.at[pl.ds(o, sz), :],
                    sem,
                )
                for o, sz in _CHUNKS
            ]

        def fire(i, rbuf, sem):
            for cp in mk_copies(i, rbuf, sem):
                cp.start()

        def drain(i, rbuf, sem):
            for cp in mk_copies(i, rbuf, sem):
                cp.wait()

        def accum_store(i, rbuf):
            def body(k, accs):
                accs = list(accs)
                for u in range(_UNROLL):
                    r = _UNROLL * k + u
                    for c in range(4):
                        accs[c] = accs[c] + rbuf[r, pl.ds(16 * c, 16)]
                return tuple(accs)

            z = jnp.zeros((16,), jnp.float32)
            accs = lax.fori_loop(0, L // _UNROLL, body, (z, z, z, z))
            for c in range(4):
                acc_v[pl.ds(16 * c, 16)] = accs[c]
            pltpu.sync_copy(acc_v, out.at[pl.ds((base + i) * EMB, EMB)])

        fire(0, rows_a, sem_a)

        def pair_body(j, _):
            i0 = 2 * j
            fire(i0 + 1, rows_b, sem_b)
            drain(i0, rows_a, sem_a)
            accum_store(i0, rows_a)

            @pl.when(j < _RW // 2 - 1)
            def _():
                fire(i0 + 2, rows_a, sem_a)

            drain(i0 + 1, rows_b, sem_b)
            accum_store(i0 + 1, rows_b)
            return 0

        lax.fori_loop(0, _RW // 2, pair_body, 0)

    return pool_kernel(x_flat, tab)


def _mlp_body(p1_ref, p2_ref, p3_ref, w1_ref, b1_ref, w2_ref, b2_ref, o_ref):
    h = jnp.dot(p1_ref[...], w1_ref[pl.ds(0, EMB), :],
                preferred_element_type=jnp.float32)
    h = h + jnp.dot(p2_ref[...], w1_ref[pl.ds(EMB, EMB), :],
                    preferred_element_type=jnp.float32)
    h = h + jnp.dot(p3_ref[...], w1_ref[pl.ds(2 * EMB, EMB), :],
                    preferred_element_type=jnp.float32)
    h = h * (1.0 / L) + b1_ref[...]
    h = jnp.maximum(h, 0.0)
    o = jnp.dot(h, w2_ref[...], preferred_element_type=jnp.float32)
    o_ref[...] = o + b2_ref[...]


def _tc_mlp(p1, p2, p3, W1p, b1, W2, b2):
    blk = 512
    grid = (B // blk,)
    return pl.pallas_call(
        _mlp_body,
        grid=grid,
        in_specs=[
            pl.BlockSpec((blk, EMB), lambda i: (i, 0)),
            pl.BlockSpec((blk, EMB), lambda i: (i, 0)),
            pl.BlockSpec((blk, EMB), lambda i: (i, 0)),
            pl.BlockSpec((POOL_W, HID), lambda i: (0, 0)),
            pl.BlockSpec((1, HID), lambda i: (0, 0)),
            pl.BlockSpec((HID, NCLS), lambda i: (0, 0)),
            pl.BlockSpec((1, NCLS), lambda i: (0, 0)),
        ],
        out_specs=pl.BlockSpec((blk, NCLS), lambda i: (i, 0)),
        out_shape=jax.ShapeDtypeStruct((B, NCLS), jnp.float32),
    )(p1, p2, p3, W1p, b1.reshape(1, HID), W2, b2.reshape(1, NCLS))


def kernel(x_word, x_bigram, x_trigram, emb_word, emb_bigram, emb_trigram,
           W1, b1, W2, b2):
    pools = []
    for x, tab in ((x_word, emb_word), (x_bigram, emb_bigram),
                   (x_trigram, emb_trigram)):
        pools.append(_sc_pool_one(x.reshape(B * L), tab))
    p1, p2, p3 = (p.reshape(B, EMB) for p in pools)
    return _tc_mlp(p1, p2, p3, W1, b1, W2, b2)


# speedup vs baseline: 2.7509x; 1.0002x over previous
import functools

import jax
import jax.numpy as jnp
from jax import lax
from jax.experimental import pallas as pl
from jax.experimental.pallas import tpu as pltpu
from jax.experimental.pallas import tpu_sc as plsc

B = 4096
L = 200
EMB = 64
HID = 256
NCLS = 10
POOL_W = 3 * EMB

_NC = 2
_NS = 16
_NW = _NC * _NS
_RW = B // _NW
_CHUNKS = ((0, 128), (128, 72))
_UNROLL = 8

def _sc_pool_one(x_flat, tab):
    mesh = plsc.VectorSubcoreMesh(core_axis_name="c", subcore_axis_name="s")

    @functools.partial(
        pl.kernel,
        mesh=mesh,
        compiler_params=pltpu.CompilerParams(use_tc_tiling_on_sc=False),
        out_type=jax.ShapeDtypeStruct((B * EMB,), jnp.float32),
        scratch_types=[
            pltpu.VMEM((_RW * L,), jnp.int32),
            pltpu.VMEM((L, EMB), jnp.float32),
            pltpu.VMEM((L, EMB), jnp.float32),
            pltpu.VMEM((EMB,), jnp.float32),
            pltpu.SemaphoreType.DMA,
            pltpu.SemaphoreType.DMA,
        ],
    )
    def pool_kernel(x_hbm, tab_hbm, out, idx_v, rows_a, rows_b, acc_v,
                    sem_a, sem_b):
        wid = lax.axis_index("s") * _NC + lax.axis_index("c")
        base = wid * _RW

        pltpu.sync_copy(x_hbm.at[pl.ds(base * L, _RW * L)], idx_v)

        def mk_copies(i, rbuf, sem):
            return [
                pltpu.make_async_copy(
                    tab_hbm.at[idx_v.at[pl.ds(i * L + o, sz)]],
                    rbuf.at[pl.ds(o, sz), :],
                    sem,
                )
                for o, sz in _CHUNKS
            ]

        def fire(i, rbuf, sem):
            for cp in mk_copies(i, rbuf, sem):
                cp.start()

        def drain(i, rbuf, sem):
            for cp in mk_copies(i, rbuf, sem):
                cp.wait()

        def accum_store(i, rbuf):
            def body(k, accs):
                accs = list(accs)
                for u in range(_UNROLL):
                    r = _UNROLL * k + u
                    for c in range(4):
                        accs[c] = accs[c] + rbuf[r, pl.ds(16 * c, 16)]
                return tuple(accs)

            z = jnp.zeros((16,), jnp.float32)
            accs = lax.fori_loop(0, L // _UNROLL, body, (z, z, z, z))
            for c in range(4):
                acc_v[pl.ds(16 * c, 16)] = accs[c]
            pltpu.sync_copy(acc_v, out.at[pl.ds((base + i) * EMB, EMB)])

        fire(0, rows_a, sem_a)

        def pair_body(j, _):
            i0 = 2 * j
            fire(i0 + 1, rows_b, sem_b)
            drain(i0, rows_a, sem_a)
            accum_store(i0, rows_a)

            @pl.when(j < _RW // 2 - 1)
            def _():
                fire(i0 + 2, rows_a, sem_a)

            drain(i0 + 1, rows_b, sem_b)
            accum_store(i0 + 1, rows_b)
            return 0

        lax.fori_loop(0, _RW // 2, pair_body, 0)

    return pool_kernel(x_flat, tab)


def _mlp_body(p1_ref, p2_ref, p3_ref, w1_ref, b1_ref, w2_ref, b2_ref, o_ref):
    h = jnp.dot(p1_ref[...], w1_ref[pl.ds(0, EMB), :],
                preferred_element_type=jnp.float32)
    h = h + jnp.dot(p2_ref[...], w1_ref[pl.ds(EMB, EMB), :],
                    preferred_element_type=jnp.float32)
    h = h + jnp.dot(p3_ref[...], w1_ref[pl.ds(2 * EMB, EMB), :],
                    preferred_element_type=jnp.float32)
    h = h * (1.0 / L) + b1_ref[...]
    h = jnp.maximum(h, 0.0)
    o = jnp.dot(h, w2_ref[...], preferred_element_type=jnp.float32)
    o_ref[...] = o + b2_ref[...]


def _tc_mlp(p1, p2, p3, W1p, b1, W2, b2):
    blk = 512
    grid = (B // blk,)
    return pl.pallas_call(
        _mlp_body,
        grid=grid,
        in_specs=[
            pl.BlockSpec((blk, EMB), lambda i: (i, 0)),
            pl.BlockSpec((blk, EMB), lambda i: (i, 0)),
            pl.BlockSpec((blk, EMB), lambda i: (i, 0)),
            pl.BlockSpec((POOL_W, HID), lambda i: (0, 0)),
            pl.BlockSpec((1, HID), lambda i: (0, 0)),
            pl.BlockSpec((HID, NCLS), lambda i: (0, 0)),
            pl.BlockSpec((1, NCLS), lambda i: (0, 0)),
        ],
        out_specs=pl.BlockSpec((blk, NCLS), lambda i: (i, 0)),
        out_shape=jax.ShapeDtypeStruct((B, NCLS), jnp.float32),
    )(p1, p2, p3, W1p, b1.reshape(1, HID), W2, b2.reshape(1, NCLS))


def kernel(x_word, x_bigram, x_trigram, emb_word, emb_bigram, emb_trigram,
           W1, b1, W2, b2):
    p2 = _sc_pool_one(x_bigram.reshape(B * L), emb_bigram)
    p3 = _sc_pool_one(x_trigram.reshape(B * L), emb_trigram)
    p1 = _sc_pool_one(x_word.reshape(B * L), emb_word)
    p1, p2, p3 = (p.reshape(B, EMB) for p in (p1, p2, p3))
    return _tc_mlp(p1, p2, p3, W1, b1, W2, b2)
